# baseline (speedup 1.0000x reference)
import functools

import jax
import jax.numpy as jnp
from jax.experimental import pallas as pl


def _layer_a_body(adj_ref, h_ref, hblk_ref, cvec_ref, w1_ref, b1_ref,
                  w2_ref, b2_ref, x_ref, stats_ref):
    pooled = jnp.dot(adj_ref[...], h_ref[...],
                     preferred_element_type=jnp.float32)
    pooled = pooled + cvec_ref[...] * hblk_ref[...]
    t = jnp.dot(pooled, w1_ref[...], preferred_element_type=jnp.float32)
    t = jnp.maximum(t + b1_ref[...], 0.0)
    x = jnp.dot(t, w2_ref[...], preferred_element_type=jnp.float32)
    x = x + b2_ref[...]
    x_ref[...] = x
    stats_ref[...] = jnp.stack([jnp.sum(x, axis=0),
                                jnp.sum(x * x, axis=0)])[None]


def _layer_a(adj, h, cvec, w1, b1, w2, b2, bm):
    n, d = h.shape
    nb = n // bm
    grid = (nb,)
    x, stats = pl.pallas_call(
        _layer_a_body,
        grid=grid,
        in_specs=[
            pl.BlockSpec((bm, n), lambda i: (i, 0)),
            pl.BlockSpec((n, d), lambda i: (0, 0)),
            pl.BlockSpec((bm, d), lambda i: (i, 0)),
            pl.BlockSpec((1, d), lambda i: (0, 0)),
            pl.BlockSpec((d, d), lambda i: (0, 0)),
            pl.BlockSpec((1, d), lambda i: (0, 0)),
            pl.BlockSpec((d, d), lambda i: (0, 0)),
            pl.BlockSpec((1, d), lambda i: (0, 0)),
        ],
        out_specs=[
            pl.BlockSpec((bm, d), lambda i: (i, 0)),
            pl.BlockSpec((1, 2, d), lambda i: (i, 0, 0)),
        ],
        out_shape=[
            jax.ShapeDtypeStruct((n, d), jnp.float32),
            jax.ShapeDtypeStruct((nb, 2, d), jnp.float32),
        ],
    )(adj, h, h, cvec, w1, b1, w2, b2)
    return x, stats


def _bn_body(x_ref, stats_ref, gamma_ref, beta_ref, h_ref):
    n = x_ref.shape[0]
    s = jnp.sum(stats_ref[...], axis=0)
    m = s[0:1] * (1.0 / n)
    var = s[1:2] * (1.0 / n) - m * m
    inv = gamma_ref[...] * jax.lax.rsqrt(var + 1e-5)
    h_ref[...] = jnp.maximum((x_ref[...] - m) * inv + beta_ref[...], 0.0)


def _bn_head_body(x_ref, stats_ref, gamma_ref, beta_ref, wp_ref, bp_ref,
                  out_ref):
    n = x_ref.shape[0]
    s = jnp.sum(stats_ref[...], axis=0)
    m = s[0:1] * (1.0 / n)
    var = s[1:2] * (1.0 / n) - m * m
    inv = gamma_ref[...] * jax.lax.rsqrt(var + 1e-5)
    h = jnp.maximum((x_ref[...] - m) * inv + beta_ref[...], 0.0)
    out_ref[...] = jnp.dot(h, wp_ref[...],
                           preferred_element_type=jnp.float32) + bp_ref[...]


def kernel(seq1, adj, W1, b1, W2, b2, gamma, beta, eps, Wp, bp):
    n, d = seq1.shape
    num_layers = W1.shape[0]
    bm = 400 if n % 400 == 0 else n

    h = seq1
    x = None
    stats = None
    for i in range(num_layers):
        cvec = jnp.broadcast_to(1.0 + eps[i], (1, d)).astype(jnp.float32)
        x, stats = _layer_a(adj, h, cvec, W1[i], b1[i].reshape(1, d),
                            W2[i], b2[i].reshape(1, d), bm)
        g = gamma[i].reshape(1, d)
        bt = beta[i].reshape(1, d)
        if i + 1 < num_layers:
            h = pl.pallas_call(
                _bn_body,
                out_shape=jax.ShapeDtypeStruct((n, d), jnp.float32),
            )(x, stats, g, bt)

    out = pl.pallas_call(
        _bn_head_body,
        out_shape=jax.ShapeDtypeStruct((n, 1), jnp.float32),
    )(x, stats, gamma[num_layers - 1].reshape(1, d),
      beta[num_layers - 1].reshape(1, d), Wp, bp.reshape(1, 1))
    return out
